# baseline (speedup 1.0000x reference)
import jax
import jax.numpy as jnp
from jax import lax
from jax.experimental import pallas as pl
from jax.experimental.pallas import tpu as pltpu
from jax.experimental.pallas import tpu_sc as plsc

DIM = 64
PAIR_DIM = 128
WINDOW = 128
NC = 2
NS = 16
NW = NC * NS
PHASES = 10


def _gather_kernel(num_indices):
    chunks = num_indices // WINDOW
    cpw = chunks // NW
    mesh = plsc.VectorSubcoreMesh(core_axis_name="c", subcore_axis_name="s")

    @pl.kernel(
        out_type=jax.ShapeDtypeStruct((num_indices, PAIR_DIM), jnp.float32),
        mesh=mesh,
        scratch_types=[
            pltpu.VMEM((cpw, WINDOW), jnp.int32),
            pltpu.VMEM((WINDOW, PAIR_DIM), jnp.float32),
            pltpu.SemaphoreType.DMA,
        ],
    )
    def kern(table_hbm, idx_hbm, out_hbm, idx_v, rows_v, sem):
        wid = lax.axis_index("s") * NC + lax.axis_index("c")
        pltpu.sync_copy(idx_hbm.at[wid], idx_v)

        @pl.loop(0, cpw)
        def _(j):
            pltpu.async_copy(table_hbm.at[idx_v.at[j]], rows_v, sem).wait()
            base = (wid * cpw + j) * WINDOW
            pltpu.sync_copy(rows_v, out_hbm.at[pl.ds(base, WINDOW)])

    return kern


def _tail_first_kernel(res_ref, xt_ref, o_ref):
    data = res_ref[...]
    par = (xt_ref[0, 0] & 1)[:, None] == 1
    sel = jnp.where(par, data[:, DIM:], data[:, :DIM])
    o_ref[0] = sel.T


def _tail_next_kernel(prev_ref, res_ref, xt_ref, o_ref):
    del prev_ref
    _tail_first_kernel(res_ref, xt_ref, o_ref)


def _tail(res, xt, prev, b, s, sp, s0):
    out_shape = jax.ShapeDtypeStruct((s, DIM, b), jnp.float32)
    res_spec = pl.BlockSpec((b, PAIR_DIM), lambda i: (i, 0))
    xt_spec = pl.BlockSpec((1, 1, b), lambda i: (i, 0, 0))
    out_spec = pl.BlockSpec((1, DIM, b), lambda i: (i + s0, 0, 0))
    if prev is None:
        return pl.pallas_call(
            _tail_first_kernel,
            grid=(sp,),
            in_specs=[res_spec, xt_spec],
            out_specs=out_spec,
            out_shape=out_shape,
        )(res, xt)
    return pl.pallas_call(
        _tail_next_kernel,
        grid=(sp,),
        in_specs=[pl.BlockSpec(memory_space=pltpu.MemorySpace.HBM), res_spec, xt_spec],
        out_specs=out_spec,
        out_shape=out_shape,
        input_output_aliases={0: 0},
    )(prev, res, xt)


def kernel(x, weight):
    b, s = x.shape
    table = weight.reshape(weight.shape[0] // 2, PAIR_DIM)
    sp = s // PHASES
    np_idx = b * sp
    gather = _gather_kernel(np_idx)
    xts = []
    ress = []
    for k in range(PHASES):
        xt_k = x[:, k * sp:(k + 1) * sp].T.astype(jnp.int32)
        idx_k = (xt_k >> 1).reshape(NW, np_idx // (NW * WINDOW), WINDOW)
        xts.append(xt_k)
        ress.append(gather(table, idx_k))
    out = None
    for k in range(PHASES):
        out = _tail(ress[k], xts[k].reshape(sp, 1, b), out, b, s, sp, k * sp)
    return jnp.transpose(out, (2, 0, 1))


# speedup vs baseline: 1.0929x; 1.0929x over previous
import jax
import jax.numpy as jnp
from jax import lax
from jax.experimental import pallas as pl
from jax.experimental.pallas import tpu as pltpu
from jax.experimental.pallas import tpu_sc as plsc

DIM = 64
PAIR_DIM = 128
WINDOW = 128
NC = 2
NS = 16
NW = NC * NS
PHASES = 5


def _gather_kernel(num_indices):
    chunks = num_indices // WINDOW
    cpw = chunks // NW
    mesh = plsc.VectorSubcoreMesh(core_axis_name="c", subcore_axis_name="s")

    @pl.kernel(
        out_type=jax.ShapeDtypeStruct((num_indices, PAIR_DIM), jnp.float32),
        mesh=mesh,
        scratch_types=[
            pltpu.VMEM((cpw, WINDOW), jnp.int32),
            pltpu.VMEM((WINDOW, PAIR_DIM), jnp.float32),
            pltpu.VMEM((WINDOW, PAIR_DIM), jnp.float32),
            pltpu.SemaphoreType.DMA,
            pltpu.SemaphoreType.DMA,
            pltpu.SemaphoreType.DMA,
        ],
    )
    def kern(table_hbm, idx_hbm, out_hbm, idx_v, rows_a, rows_b, gsem, wsem_a, wsem_b):
        wid = lax.axis_index("s") * NC + lax.axis_index("c")
        pltpu.sync_copy(idx_hbm.at[wid], idx_v)
        bufs = (rows_a, rows_b)
        wsems = (wsem_a, wsem_b)

        @pl.loop(0, cpw, step=2)
        def _(j):
            for t in range(2):
                jj = j + t
                base = (wid * cpw + jj) * WINDOW
                dst = out_hbm.at[pl.ds(base, WINDOW)]

                @pl.when(jj >= 2)
                def _():
                    pltpu.make_async_copy(bufs[t], dst, wsems[t]).wait()

                pltpu.async_copy(table_hbm.at[idx_v.at[jj]], bufs[t], gsem).wait()
                pltpu.async_copy(bufs[t], dst, wsems[t])

        for t in range(2):
            base = (wid * cpw + (cpw - 2 + t)) * WINDOW
            pltpu.make_async_copy(
                bufs[t], out_hbm.at[pl.ds(base, WINDOW)], wsems[t]
            ).wait()

    return kern


def _tail_first_kernel(res_ref, xt_ref, o_ref):
    data = res_ref[...]
    par = (xt_ref[0, 0] & 1)[:, None] == 1
    sel = jnp.where(par, data[:, DIM:], data[:, :DIM])
    o_ref[0] = sel.T


def _tail_next_kernel(prev_ref, res_ref, xt_ref, o_ref):
    del prev_ref
    _tail_first_kernel(res_ref, xt_ref, o_ref)


def _tail(res, xt, prev, b, s, sp, s0):
    out_shape = jax.ShapeDtypeStruct((s, DIM, b), jnp.float32)
    res_spec = pl.BlockSpec((b, PAIR_DIM), lambda i: (i, 0))
    xt_spec = pl.BlockSpec((1, 1, b), lambda i: (i, 0, 0))
    out_spec = pl.BlockSpec((1, DIM, b), lambda i: (i + s0, 0, 0))
    if prev is None:
        return pl.pallas_call(
            _tail_first_kernel,
            grid=(sp,),
            in_specs=[res_spec, xt_spec],
            out_specs=out_spec,
            out_shape=out_shape,
        )(res, xt)
    return pl.pallas_call(
        _tail_next_kernel,
        grid=(sp,),
        in_specs=[pl.BlockSpec(memory_space=pltpu.MemorySpace.HBM), res_spec, xt_spec],
        out_specs=out_spec,
        out_shape=out_shape,
        input_output_aliases={0: 0},
    )(prev, res, xt)


def kernel(x, weight):
    b, s = x.shape
    table = weight.reshape(weight.shape[0] // 2, PAIR_DIM)
    sp = s // PHASES
    np_idx = b * sp
    gather = _gather_kernel(np_idx)
    xts = []
    ress = []
    for k in range(PHASES):
        xt_k = x[:, k * sp:(k + 1) * sp].T.astype(jnp.int32)
        idx_k = (xt_k >> 1).reshape(NW, np_idx // (NW * WINDOW), WINDOW)
        xts.append(xt_k)
        ress.append(gather(table, idx_k))
    out = None
    for k in range(PHASES):
        out = _tail(ress[k], xts[k].reshape(sp, 1, b), out, b, s, sp, k * sp)
    return jnp.transpose(out, (2, 0, 1))
